# baseline (speedup 1.0000x reference)
import jax
import jax.numpy as jnp
from jax import lax
from jax.experimental import pallas as pl
from jax.experimental.pallas import tpu as pltpu
from jax.experimental.pallas import tpu_sc as plsc

D = 64
B = 16384
NC = 2
NS = 16
NW = NC * NS
BPW = B // NW
NCHUNK = BPW // 128
NGRP = BPW // 16


def _nrsqrt(s):
    i = plsc.bitcast(s, jnp.int32)
    i = jnp.int32(0x5F3759DF) - lax.shift_right_logical(i, jnp.int32(1))
    y = plsc.bitcast(i, jnp.float32)
    for _ in range(3):
        y = y * (jnp.float32(1.5) - jnp.float32(0.5) * s * y * y)
    return y


def _sc_body(weight_hbm, idx_hbm, ia_hbm, out_hbm,
             idx_v, ia_v, rows_v, a_v, out_v, sem):
    wid = lax.axis_index("s") * NC + lax.axis_index("c")

    pltpu.sync_copy(idx_hbm.at[pl.ds(wid * NCHUNK, NCHUNK)], idx_v)
    pltpu.sync_copy(ia_hbm, ia_v)

    handles = [pltpu.async_copy(weight_hbm.at[ia_v], a_v, sem)]
    for j in range(NCHUNK):
        handles.append(pltpu.async_copy(weight_hbm.at[idx_v.at[j]],
                                        rows_v.at[pl.ds(j * 128, 128)], sem))
    for h in handles:
        h.wait()

    a_regs = [a_v[0, pl.ds(k * 16, 16)] for k in range(D // 16)]
    sa = jnp.float32(0)
    for k in range(D // 16):
        sq = a_regs[k] * a_regs[k]
        for l in range(16):
            sa = sa + sq[l]
    sa = jnp.maximum(sa, jnp.float32(1e-16))

    lanes = lax.iota(jnp.int32, 16)

    def group(g, carry):
        row_idx = g * 16 + lanes
        acc_dot = jnp.zeros((16,), jnp.float32)
        acc_sq = jnp.zeros((16,), jnp.float32)
        for d in range(D):
            col = jnp.full((16,), d, jnp.int32)
            vals = plsc.load_gather(rows_v, [row_idx, col])
            a_d = a_regs[d // 16][d % 16]
            acc_dot = acc_dot + a_d * vals
            acc_sq = acc_sq + vals * vals
        r = _nrsqrt(sa * jnp.maximum(acc_sq, jnp.float32(1e-16)))
        out_v[pl.ds(g * 16, 16)] = acc_dot * r
        return carry

    lax.fori_loop(0, NGRP, group, None)

    pltpu.sync_copy(out_v, out_hbm.at[pl.ds(wid * BPW, BPW)])


def kernel(x, weight):
    idx = x[:, 1].astype(jnp.int32).reshape(NW * NCHUNK, 128)
    ia = jnp.broadcast_to(x[0, 0].astype(jnp.int32)[None], (8,))
    run = pl.kernel(
        _sc_body,
        out_type=jax.ShapeDtypeStruct((B,), jnp.float32),
        mesh=plsc.VectorSubcoreMesh(core_axis_name="c", subcore_axis_name="s",
                                    num_cores=NC, num_subcores=NS),
        compiler_params=pltpu.CompilerParams(needs_layout_passes=False,
                                             use_tc_tiling_on_sc=False),
        scratch_types=[
            pltpu.VMEM((NCHUNK, 128), jnp.int32),
            pltpu.VMEM((8,), jnp.int32),
            pltpu.VMEM((BPW, D), jnp.float32),
            pltpu.VMEM((8, D), jnp.float32),
            pltpu.VMEM((BPW,), jnp.float32),
            pltpu.SemaphoreType.DMA,
        ],
    )
    return run(weight, idx, ia)
